# Initial kernel scaffold; baseline (speedup 1.0000x reference)
import jax
import jax.numpy as jnp
from jax.experimental import pallas as pl


def kernel(raw_cls_bbox, raw_prob):
    raise NotImplementedError("write your pallas kernel here")


# speedup vs baseline: 141.2919x; 141.2919x over previous
import functools

import jax
import jax.numpy as jnp
from jax import lax
from jax.experimental import pallas as pl
from jax.experimental.pallas import tpu as pltpu
from jax.experimental.pallas import tpu_sc as plsc

N = 5000
N_CLASS = 21
K_KEEP = 100
NMS_THRESH = 0.3
SCORE_THRESH = 0.05

L = 16


def _permute(v, p):
    return lax.gather(
        v, p[:, None],
        lax.GatherDimensionNumbers(offset_dims=(), collapsed_slice_dims=(0,),
                                   start_index_map=(0,)),
        (1,), mode=lax.GatherScatterMode.PROMISE_IN_BOUNDS)
NC = 2
NS = 16
NPAD = 5008
NCHUNK = NPAD // L
OUT_W = 512


def _nms_body(y1_hbm, x1_hbm, y2_hbm, x2_hbm, s_hbm, out_hbm,
              y1v, x1v, y2v, x2v, sv, outv):
    wid = lax.axis_index("s") * NC + lax.axis_index("c")
    cls = jnp.minimum(wid, N_CLASS - 2)

    if True:
        pltpu.sync_copy(y1_hbm.at[cls], y1v)
        pltpu.sync_copy(x1_hbm.at[cls], x1v)
        pltpu.sync_copy(y2_hbm.at[cls], y2v)
        pltpu.sync_copy(x2_hbm.at[cls], x2v)
        pltpu.sync_copy(s_hbm.at[cls], sv)

        lanes = lax.iota(jnp.int32, L)
        zeros = jnp.zeros((L,), jnp.float32)

        def init_out(i, _):
            outv[pl.ds(i * L, L)] = zeros
            return 0

        lax.fori_loop(0, OUT_W // L, init_out, 0)

        def scan_pass(py1, px1, py2, px2, parea):
            def chunk(j, carry):
                bval, bidx = carry
                sl = pl.ds(j * L, L)
                cy1 = y1v[sl]
                cx1 = x1v[sl]
                cy2 = y2v[sl]
                cx2 = x2v[sl]
                cs = sv[sl]
                ty = jnp.maximum(cy1, py1)
                tx = jnp.maximum(cx1, px1)
                by = jnp.minimum(cy2, py2)
                bx = jnp.minimum(cx2, px2)
                inter = jnp.maximum(by - ty, 0.0) * jnp.maximum(bx - tx, 0.0)
                carea = jnp.maximum(cy2 - cy1, 0.0) * jnp.maximum(cx2 - cx1, 0.0)
                iou = inter / (parea + carea - inter + 1e-9)
                cs = jnp.where(iou > NMS_THRESH, -1.0, cs)
                sv[sl] = cs
                upd = cs > bval
                bval = jnp.where(upd, cs, bval)
                bidx = jnp.where(upd, j * L + lanes, bidx)
                return bval, bidx

            neg = jnp.full((L,), -2.0, jnp.float32)
            return lax.fori_loop(0, NCHUNK, chunk, (neg, jnp.zeros((L,), jnp.int32)))

        perms = [jnp.bitwise_xor(lanes, s) for s in (8, 4, 2, 1)]

        def body(k, carry):
            py1, px1, py2, px2, parea = carry
            bval, bidx = scan_pass(py1, px1, py2, px2, parea)
            vmax = bval
            for p in perms:
                vmax = jnp.maximum(vmax, _permute(vmax, p))
            cand = jnp.where(bval == vmax, bidx, jnp.int32(NPAD))
            for p in perms:
                cand = jnp.minimum(cand, _permute(cand, p))
            midx = cand
            ny1 = plsc.load_gather(y1v, [midx])
            nx1 = plsc.load_gather(x1v, [midx])
            ny2 = plsc.load_gather(y2v, [midx])
            nx2 = plsc.load_gather(x2v, [midx])
            narea = jnp.maximum(ny2 - ny1, 0.0) * jnp.maximum(nx2 - nx1, 0.0)
            found = vmax > SCORE_THRESH
            val = jnp.where(lanes == 0, ny1,
                  jnp.where(lanes == 1, nx1,
                  jnp.where(lanes == 2, ny2,
                  jnp.where(lanes == 3, nx2, vmax))))
            omask = (lanes < 5) & found
            plsc.store_scatter(outv, [k * 5 + lanes], val, mask=omask)
            plsc.store_scatter(sv, [midx], jnp.full((L,), -1.0),
                               mask=lanes == 0)
            return ny1, nx1, ny2, nx2, narea

        z = jnp.zeros((L,), jnp.float32)
        lax.fori_loop(0, K_KEEP, body, (z, z, z, z, z))

        @pl.when(wid < N_CLASS - 1)
        def _():
            pltpu.sync_copy(outv, out_hbm.at[cls])


@functools.partial(jax.jit, static_argnums=())
def _sc_nms(y1, x1, y2, x2, s):
    mesh = plsc.VectorSubcoreMesh(core_axis_name="c", subcore_axis_name="s")
    f = pl.kernel(
        _nms_body,
        out_type=jax.ShapeDtypeStruct((N_CLASS - 1, OUT_W), jnp.float32),
        mesh=mesh,
        compiler_params=pltpu.CompilerParams(needs_layout_passes=False),
        scratch_types=[
            pltpu.VMEM((NPAD,), jnp.float32),
            pltpu.VMEM((NPAD,), jnp.float32),
            pltpu.VMEM((NPAD,), jnp.float32),
            pltpu.VMEM((NPAD,), jnp.float32),
            pltpu.VMEM((NPAD,), jnp.float32),
            pltpu.VMEM((OUT_W,), jnp.float32),
        ],
    )
    return f(y1, x1, y2, x2, s)


def kernel(raw_cls_bbox, raw_prob):
    cls_bbox = raw_cls_bbox.reshape(N, N_CLASS, 4)
    comps = jnp.transpose(cls_bbox, (1, 2, 0))[1:]
    comps = jnp.pad(comps, ((0, 0), (0, 0), (0, NPAD - N)))
    scores = raw_prob.T[1:]
    scores = jnp.pad(scores, ((0, 0), (0, NPAD - N)), constant_values=-1.0)
    out = _sc_nms(comps[:, 0], comps[:, 1], comps[:, 2], comps[:, 3], scores)
    return out[:, :5 * K_KEEP].reshape(N_CLASS - 1, K_KEEP, 5)


# speedup vs baseline: 574.4432x; 4.0657x over previous
import functools

import jax
import jax.numpy as jnp
from jax import lax
from jax.experimental import pallas as pl
from jax.experimental.pallas import tpu as pltpu
from jax.experimental.pallas import tpu_sc as plsc

N = 5000
N_CLASS = 21
K_KEEP = 100
NMS_THRESH = 0.3
SCORE_THRESH = 0.05

L = 16
NC = 2
NS = 16
NPAD = 5008
NCHUNK = NPAD // L
OUT_W = 512
RC = 10
SUPER = K_KEEP // RC


def _permute(v, p):
    return lax.gather(
        v, p[:, None],
        lax.GatherDimensionNumbers(offset_dims=(), collapsed_slice_dims=(0,),
                                   start_index_map=(0,)),
        (1,), mode=lax.GatherScatterMode.PROMISE_IN_BOUNDS)


def _nms_body(y1_hbm, x1_hbm, y2_hbm, x2_hbm, s_hbm, out_hbm,
              y1v, x1v, y2v, x2v, sv, outv, cntv):
    wid = lax.axis_index("s") * NC + lax.axis_index("c")
    cls = jnp.minimum(wid, N_CLASS - 2)

    pltpu.sync_copy(y1_hbm.at[cls], y1v)
    pltpu.sync_copy(x1_hbm.at[cls], x1v)
    pltpu.sync_copy(y2_hbm.at[cls], y2v)
    pltpu.sync_copy(x2_hbm.at[cls], x2v)
    pltpu.sync_copy(s_hbm.at[cls], sv)

    lanes = lax.iota(jnp.int32, L)
    last = jnp.full((L,), L - 1, jnp.int32)
    zeros = jnp.zeros((L,), jnp.float32)
    neg1 = jnp.full((L,), -1.0, jnp.float32)
    perms = [jnp.bitwise_xor(lanes, s) for s in (8, 4, 2, 1)]

    def init_out(i, _):
        outv[pl.ds(i * L, L)] = zeros
        return 0

    lax.fori_loop(0, OUT_W // L, init_out, 0)

    def compact(nch):
        @pl.loop(0, nch, init_carry=jnp.zeros((L,), jnp.int32))
        def basev(j, base):
            sl = pl.ds(j * L, L)
            cy1 = y1v[sl]
            cx1 = x1v[sl]
            cy2 = y2v[sl]
            cx2 = x2v[sl]
            cs = sv[sl]
            alive = cs > SCORE_THRESH
            ai = alive.astype(jnp.int32)
            incl = plsc.cumsum(ai)
            pos = base + (incl - ai)
            plsc.store_scatter(y1v, [pos], cy1, mask=alive)
            plsc.store_scatter(x1v, [pos], cx1, mask=alive)
            plsc.store_scatter(y2v, [pos], cy2, mask=alive)
            plsc.store_scatter(x2v, [pos], cx2, mask=alive)
            plsc.store_scatter(sv, [pos], cs, mask=alive)
            return base + _permute(incl, last)

        cntv[pl.ds(0, L)] = basev
        n2 = cntv[pl.ds(0, L)][0]
        pad = (((n2 + L - 1) >> 4) << 4) - n2
        plsc.store_scatter(sv, [n2 + lanes], neg1, mask=lanes < pad)
        return n2

    n_alive = compact(jnp.int32(NCHUNK))

    def super_body(t, carry):
        py1, px1, py2, px2, parea, n = carry
        nch = (n + L - 1) >> 4

        def body(k2, c2):
            py1, px1, py2, px2, parea = c2
            k = t * RC + k2

            @pl.loop(0, nch,
                     init_carry=(jnp.full((L,), -2.0, jnp.float32),
                                 jnp.zeros((L,), jnp.int32)))
            def scanned(j, sc_carry):
                bval, bidx = sc_carry
                sl = pl.ds(j * L, L)
                cy1 = y1v[sl]
                cx1 = x1v[sl]
                cy2 = y2v[sl]
                cx2 = x2v[sl]
                cs = sv[sl]
                ty = jnp.maximum(cy1, py1)
                tx = jnp.maximum(cx1, px1)
                by = jnp.minimum(cy2, py2)
                bx = jnp.minimum(cx2, px2)
                inter = jnp.maximum(by - ty, 0.0) * jnp.maximum(bx - tx, 0.0)
                carea = jnp.maximum(cy2 - cy1, 0.0) * jnp.maximum(cx2 - cx1, 0.0)
                iou = inter / (parea + carea - inter + 1e-9)
                cs = jnp.where(iou > NMS_THRESH, -1.0, cs)
                sv[sl] = cs
                upd = cs > bval
                bval = jnp.where(upd, cs, bval)
                bidx = jnp.where(upd, j * L + lanes, bidx)
                return bval, bidx

            bval, bidx = scanned
            vmax = bval
            for p in perms:
                vmax = jnp.maximum(vmax, _permute(vmax, p))
            cand = jnp.where(bval == vmax, bidx, jnp.int32(NPAD))
            for p in perms:
                cand = jnp.minimum(cand, _permute(cand, p))
            midx = cand
            ny1 = plsc.load_gather(y1v, [midx])
            nx1 = plsc.load_gather(x1v, [midx])
            ny2 = plsc.load_gather(y2v, [midx])
            nx2 = plsc.load_gather(x2v, [midx])
            narea = jnp.maximum(ny2 - ny1, 0.0) * jnp.maximum(nx2 - nx1, 0.0)
            found = vmax > SCORE_THRESH
            val = jnp.where(lanes == 0, ny1,
                  jnp.where(lanes == 1, nx1,
                  jnp.where(lanes == 2, ny2,
                  jnp.where(lanes == 3, nx2, vmax))))
            omask = (lanes < 5) & found
            plsc.store_scatter(outv, [k * 5 + lanes], val, mask=omask)
            plsc.store_scatter(sv, [midx], neg1, mask=lanes == 0)
            return ny1, nx1, ny2, nx2, narea

        piv = lax.fori_loop(0, RC, body, (py1, px1, py2, px2, parea))
        n2 = compact(nch)
        return (*piv, n2)

    lax.fori_loop(0, SUPER, super_body,
                  (zeros, zeros, zeros, zeros, zeros, n_alive))

    @pl.when(wid < N_CLASS - 1)
    def _():
        pltpu.sync_copy(outv, out_hbm.at[cls])


@functools.partial(jax.jit, static_argnums=())
def _sc_nms(y1, x1, y2, x2, s):
    mesh = plsc.VectorSubcoreMesh(core_axis_name="c", subcore_axis_name="s")
    f = pl.kernel(
        _nms_body,
        out_type=jax.ShapeDtypeStruct((N_CLASS - 1, OUT_W), jnp.float32),
        mesh=mesh,
        compiler_params=pltpu.CompilerParams(needs_layout_passes=False),
        scratch_types=[
            pltpu.VMEM((NPAD,), jnp.float32),
            pltpu.VMEM((NPAD,), jnp.float32),
            pltpu.VMEM((NPAD,), jnp.float32),
            pltpu.VMEM((NPAD,), jnp.float32),
            pltpu.VMEM((NPAD,), jnp.float32),
            pltpu.VMEM((OUT_W,), jnp.float32),
            pltpu.VMEM((L,), jnp.int32),
        ],
    )
    return f(y1, x1, y2, x2, s)


def kernel(raw_cls_bbox, raw_prob):
    cls_bbox = raw_cls_bbox.reshape(N, N_CLASS, 4)
    comps = jnp.transpose(cls_bbox, (1, 2, 0))[1:]
    comps = jnp.pad(comps, ((0, 0), (0, 0), (0, NPAD - N)))
    scores = raw_prob.T[1:]
    scores = jnp.pad(scores, ((0, 0), (0, NPAD - N)), constant_values=-1.0)
    out = _sc_nms(comps[:, 0], comps[:, 1], comps[:, 2], comps[:, 3], scores)
    return out[:, :5 * K_KEEP].reshape(N_CLASS - 1, K_KEEP, 5)


# speedup vs baseline: 583.3225x; 1.0155x over previous
import functools

import jax
import jax.numpy as jnp
from jax import lax
from jax.experimental import pallas as pl
from jax.experimental.pallas import tpu as pltpu
from jax.experimental.pallas import tpu_sc as plsc

N = 5000
N_CLASS = 21
K_KEEP = 100
NMS_THRESH = 0.3
SCORE_THRESH = 0.05

L = 16
NC = 2
NS = 16
NPAD = 5008
NCHUNK = NPAD // L
OUT_W = 512
NBUF = NPAD + 2 * L
RC = 10
SUPER = K_KEEP // RC


def _permute(v, p):
    return lax.gather(
        v, p[:, None],
        lax.GatherDimensionNumbers(offset_dims=(), collapsed_slice_dims=(0,),
                                   start_index_map=(0,)),
        (1,), mode=lax.GatherScatterMode.PROMISE_IN_BOUNDS)


def _nms_body(y1_hbm, x1_hbm, y2_hbm, x2_hbm, s_hbm, out_hbm,
              y1v, x1v, y2v, x2v, sv, av, outv, cntv):
    wid = lax.axis_index("s") * NC + lax.axis_index("c")
    cls = jnp.minimum(wid, N_CLASS - 2)

    pltpu.sync_copy(y1_hbm.at[cls], y1v)
    pltpu.sync_copy(x1_hbm.at[cls], x1v)
    pltpu.sync_copy(y2_hbm.at[cls], y2v)
    pltpu.sync_copy(x2_hbm.at[cls], x2v)
    pltpu.sync_copy(s_hbm.at[cls], sv)

    lanes = lax.iota(jnp.int32, L)
    last = jnp.full((L,), L - 1, jnp.int32)
    zeros = jnp.zeros((L,), jnp.float32)
    neg1 = jnp.full((L,), -1.0, jnp.float32)
    perms = [jnp.bitwise_xor(lanes, s) for s in (8, 4, 2, 1)]

    def init_out(i, _):
        outv[pl.ds(i * L, L)] = zeros
        return 0

    lax.fori_loop(0, OUT_W // L, init_out, 0)

    def compact(nch, init_area):
        @pl.loop(0, nch, init_carry=jnp.zeros((L,), jnp.int32))
        def basev(j, base):
            sl = pl.ds(j * L, L)
            cy1 = y1v[sl]
            cx1 = x1v[sl]
            cy2 = y2v[sl]
            cx2 = x2v[sl]
            cs = sv[sl]
            if init_area:
                ca = (jnp.maximum(cy2 - cy1, 0.0) *
                      jnp.maximum(cx2 - cx1, 0.0))
            else:
                ca = av[sl]
            alive = cs > SCORE_THRESH
            ai = alive.astype(jnp.int32)
            incl = plsc.cumsum(ai)
            pos = base + (incl - ai)
            plsc.store_scatter(y1v, [pos], cy1, mask=alive)
            plsc.store_scatter(x1v, [pos], cx1, mask=alive)
            plsc.store_scatter(y2v, [pos], cy2, mask=alive)
            plsc.store_scatter(x2v, [pos], cx2, mask=alive)
            plsc.store_scatter(sv, [pos], cs, mask=alive)
            plsc.store_scatter(av, [pos], ca, mask=alive)
            return base + _permute(incl, last)

        cntv[pl.ds(0, L)] = basev
        n2 = cntv[pl.ds(0, L)][0]
        pad = (((n2 + 2 * L - 1) >> 5) << 5) - n2
        plsc.store_scatter(sv, [n2 + lanes], neg1, mask=lanes < pad)
        plsc.store_scatter(sv, [n2 + L + lanes], neg1, mask=(lanes + L) < pad)
        return n2

    n_alive = compact(jnp.int32(NBUF // L), True)

    def super_body(t, carry):
        py1, px1, py2, px2, parea, n = carry
        npair = (n + 2 * L - 1) >> 5

        def body(k2, c2):
            py1, px1, py2, px2, parea = c2
            k = t * RC + k2

            def suppress_chunk(base):
                sl = pl.ds(base, L)
                cy1 = y1v[sl]
                cx1 = x1v[sl]
                cy2 = y2v[sl]
                cx2 = x2v[sl]
                cs = sv[sl]
                carea = av[sl]
                ty = jnp.maximum(cy1, py1)
                tx = jnp.maximum(cx1, px1)
                by = jnp.minimum(cy2, py2)
                bx = jnp.minimum(cx2, px2)
                inter = jnp.maximum(by - ty, 0.0) * jnp.maximum(bx - tx, 0.0)
                iou = inter / (parea + carea - inter + 1e-9)
                cs = jnp.where(iou > NMS_THRESH, -1.0, cs)
                sv[sl] = cs
                return cs

            @pl.loop(0, npair,
                     init_carry=(jnp.full((L,), -2.0, jnp.float32),
                                 jnp.zeros((L,), jnp.int32)))
            def scanned(j, sc_carry):
                bval, bidx = sc_carry
                base = j * (2 * L)
                cs0 = suppress_chunk(base)
                cs1 = suppress_chunk(base + L)
                upd0 = cs0 > bval
                bval = jnp.where(upd0, cs0, bval)
                bidx = jnp.where(upd0, base + lanes, bidx)
                upd1 = cs1 > bval
                bval = jnp.where(upd1, cs1, bval)
                bidx = jnp.where(upd1, base + L + lanes, bidx)
                return bval, bidx

            bval, bidx = scanned
            vmax = bval
            for p in perms:
                vmax = jnp.maximum(vmax, _permute(vmax, p))
            cand = jnp.where(bval == vmax, bidx, jnp.int32(NPAD))
            for p in perms:
                cand = jnp.minimum(cand, _permute(cand, p))
            midx = cand
            ny1 = plsc.load_gather(y1v, [midx])
            nx1 = plsc.load_gather(x1v, [midx])
            ny2 = plsc.load_gather(y2v, [midx])
            nx2 = plsc.load_gather(x2v, [midx])
            narea = jnp.maximum(ny2 - ny1, 0.0) * jnp.maximum(nx2 - nx1, 0.0)
            found = vmax > SCORE_THRESH
            val = jnp.where(lanes == 0, ny1,
                  jnp.where(lanes == 1, nx1,
                  jnp.where(lanes == 2, ny2,
                  jnp.where(lanes == 3, nx2, vmax))))
            omask = (lanes < 5) & found
            plsc.store_scatter(outv, [k * 5 + lanes], val, mask=omask)
            plsc.store_scatter(sv, [midx], neg1, mask=lanes == 0)
            return ny1, nx1, ny2, nx2, narea

        piv = lax.fori_loop(0, RC, body, (py1, px1, py2, px2, parea))
        n2 = compact(npair * 2, False)
        return (*piv, n2)

    lax.fori_loop(0, SUPER, super_body,
                  (zeros, zeros, zeros, zeros, zeros, n_alive))

    @pl.when(wid < N_CLASS - 1)
    def _():
        pltpu.sync_copy(outv, out_hbm.at[cls])


@functools.partial(jax.jit, static_argnums=())
def _sc_nms(y1, x1, y2, x2, s):
    mesh = plsc.VectorSubcoreMesh(core_axis_name="c", subcore_axis_name="s")
    f = pl.kernel(
        _nms_body,
        out_type=jax.ShapeDtypeStruct((N_CLASS - 1, OUT_W), jnp.float32),
        mesh=mesh,
        compiler_params=pltpu.CompilerParams(needs_layout_passes=False),
        scratch_types=[
            pltpu.VMEM((NBUF,), jnp.float32),
            pltpu.VMEM((NBUF,), jnp.float32),
            pltpu.VMEM((NBUF,), jnp.float32),
            pltpu.VMEM((NBUF,), jnp.float32),
            pltpu.VMEM((NBUF,), jnp.float32),
            pltpu.VMEM((NBUF,), jnp.float32),
            pltpu.VMEM((OUT_W,), jnp.float32),
            pltpu.VMEM((L,), jnp.int32),
        ],
    )
    return f(y1, x1, y2, x2, s)


def kernel(raw_cls_bbox, raw_prob):
    cls_bbox = raw_cls_bbox.reshape(N, N_CLASS, 4)
    comps = jnp.transpose(cls_bbox, (1, 2, 0))[1:]
    comps = jnp.pad(comps, ((0, 0), (0, 0), (0, NBUF - N)))
    scores = raw_prob.T[1:]
    scores = jnp.pad(scores, ((0, 0), (0, NBUF - N)), constant_values=-1.0)
    out = _sc_nms(comps[:, 0], comps[:, 1], comps[:, 2], comps[:, 3], scores)
    return out[:, :5 * K_KEEP].reshape(N_CLASS - 1, K_KEEP, 5)


# speedup vs baseline: 1459.2943x; 2.5017x over previous
import functools

import jax
import jax.numpy as jnp
from jax import lax
from jax.experimental import pallas as pl
from jax.experimental.pallas import tpu as pltpu
from jax.experimental.pallas import tpu_sc as plsc

N = 5000
N_CLASS = 21
K_KEEP = 100
NMS_THRESH = 0.3
SCORE_THRESH = 0.05

L = 16
NC = 2
NS = 16
NPAD = 5008
NCHUNK = NPAD // L
OUT_W = 512
NBUF = NPAD + 2 * L
RC = 10
SUPER = K_KEEP // RC


def _permute(v, p):
    return lax.gather(
        v, p[:, None],
        lax.GatherDimensionNumbers(offset_dims=(), collapsed_slice_dims=(0,),
                                   start_index_map=(0,)),
        (1,), mode=lax.GatherScatterMode.PROMISE_IN_BOUNDS)


def _nms_body(y1_hbm, x1_hbm, y2_hbm, x2_hbm, s_hbm, out_hbm,
              y1v, x1v, y2v, x2v, sv, av, outv, cntv):
    wid = lax.axis_index("s") * NC + lax.axis_index("c")
    cls = jnp.minimum(wid, N_CLASS - 2)

    pltpu.sync_copy(y1_hbm.at[cls], y1v)
    pltpu.sync_copy(x1_hbm.at[cls], x1v)
    pltpu.sync_copy(y2_hbm.at[cls], y2v)
    pltpu.sync_copy(x2_hbm.at[cls], x2v)
    pltpu.sync_copy(s_hbm.at[cls], sv)

    lanes = lax.iota(jnp.int32, L)
    last = jnp.full((L,), L - 1, jnp.int32)
    zeros = jnp.zeros((L,), jnp.float32)
    neg1 = jnp.full((L,), -1.0, jnp.float32)
    perms = [jnp.bitwise_xor(lanes, s) for s in (8, 4, 2, 1)]

    def init_out(i, _):
        outv[pl.ds(i * L, L)] = zeros
        return 0

    lax.fori_loop(0, OUT_W // L, init_out, 0)

    def compact(nch, init_area):
        @plsc.parallel_loop(0, nch, unroll=2,
                            carry=jnp.zeros((L,), jnp.int32))
        def basev(j, base):
            sl = pl.ds(j * L, L)
            cy1 = y1v[sl]
            cx1 = x1v[sl]
            cy2 = y2v[sl]
            cx2 = x2v[sl]
            cs = sv[sl]
            if init_area:
                ca = (jnp.maximum(cy2 - cy1, 0.0) *
                      jnp.maximum(cx2 - cx1, 0.0))
            else:
                ca = av[sl]
            alive = cs > SCORE_THRESH
            ai = alive.astype(jnp.int32)
            incl = plsc.cumsum(ai)
            pos = base + (incl - ai)
            plsc.store_scatter(y1v, [pos], cy1, mask=alive)
            plsc.store_scatter(x1v, [pos], cx1, mask=alive)
            plsc.store_scatter(y2v, [pos], cy2, mask=alive)
            plsc.store_scatter(x2v, [pos], cx2, mask=alive)
            plsc.store_scatter(sv, [pos], cs, mask=alive)
            plsc.store_scatter(av, [pos], ca, mask=alive)
            return base + _permute(incl, last)

        cntv[pl.ds(0, L)] = basev
        n2 = cntv[pl.ds(0, L)][0]
        pad = (((n2 + 2 * L - 1) >> 5) << 5) - n2
        plsc.store_scatter(sv, [n2 + lanes], neg1, mask=lanes < pad)
        plsc.store_scatter(sv, [n2 + L + lanes], neg1, mask=(lanes + L) < pad)
        return n2

    n_alive = compact(jnp.int32(NBUF // L), True)

    def super_body(t, carry):
        py1, px1, py2, px2, parea, n = carry
        npair = (n + 2 * L - 1) >> 5

        def body(k2, c2):
            py1, px1, py2, px2, parea = c2
            k = t * RC + k2

            def suppress_chunk(base):
                sl = pl.ds(base, L)
                cy1 = y1v[sl]
                cx1 = x1v[sl]
                cy2 = y2v[sl]
                cx2 = x2v[sl]
                cs = sv[sl]
                carea = av[sl]
                ty = jnp.maximum(cy1, py1)
                tx = jnp.maximum(cx1, px1)
                by = jnp.minimum(cy2, py2)
                bx = jnp.minimum(cx2, px2)
                inter = jnp.maximum(by - ty, 0.0) * jnp.maximum(bx - tx, 0.0)
                iou = inter / (parea + carea - inter + 1e-9)
                cs = jnp.where(iou > NMS_THRESH, -1.0, cs)
                sv[sl] = cs
                return cs

            @plsc.parallel_loop(0, npair * 2, unroll=4,
                                carry=(jnp.full((L,), -2.0, jnp.float32),
                                       jnp.zeros((L,), jnp.int32)))
            def scanned(j, sc_carry):
                bval, bidx = sc_carry
                base = j * L
                cs0 = suppress_chunk(base)
                upd0 = cs0 > bval
                bval = jnp.where(upd0, cs0, bval)
                bidx = jnp.where(upd0, base + lanes, bidx)
                return bval, bidx

            bval, bidx = scanned
            vmax = bval
            for p in perms:
                vmax = jnp.maximum(vmax, _permute(vmax, p))
            cand = jnp.where(bval == vmax, bidx, jnp.int32(NPAD))
            for p in perms:
                cand = jnp.minimum(cand, _permute(cand, p))
            midx = cand
            ny1 = plsc.load_gather(y1v, [midx])
            nx1 = plsc.load_gather(x1v, [midx])
            ny2 = plsc.load_gather(y2v, [midx])
            nx2 = plsc.load_gather(x2v, [midx])
            narea = jnp.maximum(ny2 - ny1, 0.0) * jnp.maximum(nx2 - nx1, 0.0)
            found = vmax > SCORE_THRESH
            val = jnp.where(lanes == 0, ny1,
                  jnp.where(lanes == 1, nx1,
                  jnp.where(lanes == 2, ny2,
                  jnp.where(lanes == 3, nx2, vmax))))
            omask = (lanes < 5) & found
            plsc.store_scatter(outv, [k * 5 + lanes], val, mask=omask)
            plsc.store_scatter(sv, [midx], neg1, mask=lanes == 0)
            return ny1, nx1, ny2, nx2, narea

        piv = lax.fori_loop(0, RC, body, (py1, px1, py2, px2, parea))
        n2 = compact(npair * 2, False)
        return (*piv, n2)

    lax.fori_loop(0, SUPER, super_body,
                  (zeros, zeros, zeros, zeros, zeros, n_alive))

    @pl.when(wid < N_CLASS - 1)
    def _():
        pltpu.sync_copy(outv, out_hbm.at[cls])


@functools.partial(jax.jit, static_argnums=())
def _sc_nms(y1, x1, y2, x2, s):
    mesh = plsc.VectorSubcoreMesh(core_axis_name="c", subcore_axis_name="s")
    f = pl.kernel(
        _nms_body,
        out_type=jax.ShapeDtypeStruct((N_CLASS - 1, OUT_W), jnp.float32),
        mesh=mesh,
        compiler_params=pltpu.CompilerParams(needs_layout_passes=False),
        scratch_types=[
            pltpu.VMEM((NBUF,), jnp.float32),
            pltpu.VMEM((NBUF,), jnp.float32),
            pltpu.VMEM((NBUF,), jnp.float32),
            pltpu.VMEM((NBUF,), jnp.float32),
            pltpu.VMEM((NBUF,), jnp.float32),
            pltpu.VMEM((NBUF,), jnp.float32),
            pltpu.VMEM((OUT_W,), jnp.float32),
            pltpu.VMEM((L,), jnp.int32),
        ],
    )
    return f(y1, x1, y2, x2, s)


def kernel(raw_cls_bbox, raw_prob):
    cls_bbox = raw_cls_bbox.reshape(N, N_CLASS, 4)
    comps = jnp.transpose(cls_bbox, (1, 2, 0))[1:]
    comps = jnp.pad(comps, ((0, 0), (0, 0), (0, NBUF - N)))
    scores = raw_prob.T[1:]
    scores = jnp.pad(scores, ((0, 0), (0, NBUF - N)), constant_values=-1.0)
    out = _sc_nms(comps[:, 0], comps[:, 1], comps[:, 2], comps[:, 3], scores)
    return out[:, :5 * K_KEEP].reshape(N_CLASS - 1, K_KEEP, 5)
